# Initial kernel scaffold; baseline (speedup 1.0000x reference)
import jax
import jax.numpy as jnp
from jax.experimental import pallas as pl


def kernel(x, edge_feat, edge_index, Wq, Wk, Wv, We, Wout, bout, Wg, bg, gamma1, beta1, W1, b1, W2, b2, gamma2, beta2):
    raise NotImplementedError("write your pallas kernel here")


# speedup vs baseline: 1.2814x; 1.2814x over previous
import functools
import math

import jax
import jax.numpy as jnp
from jax.experimental import pallas as pl

N = 10000
E = 320000
D = 128
H = 8
DH = 16


def _qkv_body(x_ref, wq_ref, wk_ref, wv_ref, q_ref, kv_ref):
    xb = x_ref[...]
    q_ref[...] = jnp.dot(xb, wq_ref[...], preferred_element_type=jnp.float32)
    kv_ref[:, :D] = jnp.dot(xb, wk_ref[...], preferred_element_type=jnp.float32)
    kv_ref[:, D:] = jnp.dot(xb, wv_ref[...], preferred_element_type=jnp.float32)


def _qkv(x, Wq, Wk, Wv):
    blk = 1000
    grid = (N // blk,)
    return pl.pallas_call(
        _qkv_body,
        grid=grid,
        in_specs=[
            pl.BlockSpec((blk, D), lambda i: (i, 0)),
            pl.BlockSpec((D, D), lambda i: (0, 0)),
            pl.BlockSpec((D, D), lambda i: (0, 0)),
            pl.BlockSpec((D, D), lambda i: (0, 0)),
        ],
        out_specs=[
            pl.BlockSpec((blk, D), lambda i: (i, 0)),
            pl.BlockSpec((blk, 2 * D), lambda i: (i, 0)),
        ],
        out_shape=[
            jax.ShapeDtypeStruct((N, D), jnp.float32),
            jax.ShapeDtypeStruct((N, 2 * D), jnp.float32),
        ],
    )(x, Wq, Wk, Wv)


def _eproj_body(ef_ref, we_ref, out_ref):
    out_ref[...] = jnp.dot(ef_ref[...], we_ref[...],
                           preferred_element_type=jnp.float32)


def _eproj(edge_feat, We):
    ef8 = edge_feat.reshape(E // 8, 8 * 16)
    we_bd = jnp.zeros((8 * 16, 8 * H), jnp.float32)
    for j in range(8):
        we_bd = we_bd.at[j * 16:(j + 1) * 16, j * H:(j + 1) * H].set(We)
    blk = 4000
    grid = (E // 8 // blk,)
    out = pl.pallas_call(
        _eproj_body,
        grid=grid,
        in_specs=[
            pl.BlockSpec((blk, 128), lambda i: (i, 0)),
            pl.BlockSpec((128, 64), lambda i: (0, 0)),
        ],
        out_specs=pl.BlockSpec((blk, 64), lambda i: (i, 0)),
        out_shape=jax.ShapeDtypeStruct((E // 8, 64), jnp.float32),
    )(ef8, we_bd)
    return out.reshape(E, H)


def _edge_phase(q, kv, eproj, src, dst):
    k = kv[:, :D]
    v = kv[:, D:]
    qe = q[dst].reshape(-1, H, DH)
    ke = k[src].reshape(-1, H, DH)
    ve = v[src].reshape(-1, H, DH)
    score = jnp.sum(qe * ke, axis=-1) / math.sqrt(DH) + eproj
    alpha = jnp.exp(score)
    den = jax.ops.segment_sum(alpha, dst, num_segments=N)
    msg = (alpha[..., None] * ve).reshape(-1, H * DH)
    num = jax.ops.segment_sum(msg, dst, num_segments=N)
    return num, den


def _epi_body(num_ref, den_ref, x_ref, wout_ref, bout_ref, wg1_ref, wg2_ref,
              bg_ref, g1_ref, b1n_ref, w1_ref, bb1_ref, w2_ref, bb2_ref,
              g2_ref, b2n_ref, out_ref):
    num = num_ref[...]
    den = den_ref[...]
    xb = x_ref[...]
    rep = (jnp.arange(D)[None, :] // DH == jnp.arange(H)[:, None]
           ).astype(jnp.float32)
    den_rep = jnp.dot(den, rep, preferred_element_type=jnp.float32)
    agg = num / (den_rep + 1e-20)
    out_lin = jnp.dot(agg, wout_ref[...],
                      preferred_element_type=jnp.float32) + bout_ref[...]
    gz = (jnp.dot(out_lin, wg1_ref[...], preferred_element_type=jnp.float32)
          + jnp.dot(xb, wg2_ref[...], preferred_element_type=jnp.float32)
          + bg_ref[...])
    g = jax.nn.sigmoid(gz)
    h = g * out_lin + (1.0 - g) * xb
    mu = jnp.mean(h, axis=-1, keepdims=True)
    var = jnp.mean((h - mu) ** 2, axis=-1, keepdims=True)
    y = (h - mu) * jax.lax.rsqrt(var + 1e-5) * g1_ref[...] + b1n_ref[...]
    z = jnp.dot(y, w1_ref[...], preferred_element_type=jnp.float32) + bb1_ref[...]
    z = z * jax.nn.sigmoid(z)
    y2 = jnp.dot(z, w2_ref[...], preferred_element_type=jnp.float32) + bb2_ref[...]
    s = y + y2
    mu2 = jnp.mean(s, axis=-1, keepdims=True)
    var2 = jnp.mean((s - mu2) ** 2, axis=-1, keepdims=True)
    out_ref[...] = ((s - mu2) * jax.lax.rsqrt(var2 + 1e-5) * g2_ref[...]
                    + b2n_ref[...])


def _epilogue(num, den, x, Wout, bout, Wg, bg, gamma1, beta1, W1, b1, W2, b2,
              gamma2, beta2):
    blk = 1000
    grid = (N // blk,)
    Wg1 = Wg[:D]
    Wg2 = Wg[D:]
    row = lambda i: (i, 0)
    full = lambda r, c: pl.BlockSpec((r, c), lambda i: (0, 0))
    vec = lambda c: pl.BlockSpec((1, c), lambda i: (0, 0))
    DFF = W1.shape[1]
    return pl.pallas_call(
        _epi_body,
        grid=grid,
        in_specs=[
            pl.BlockSpec((blk, D), row),
            pl.BlockSpec((blk, H), row),
            pl.BlockSpec((blk, D), row),
            full(D, D), vec(D), full(D, D), full(D, D), vec(D),
            vec(D), vec(D), full(D, DFF), vec(DFF), full(DFF, D), vec(D),
            vec(D), vec(D),
        ],
        out_specs=pl.BlockSpec((blk, D), row),
        out_shape=jax.ShapeDtypeStruct((N, D), jnp.float32),
    )(num, den, x, Wout, bout.reshape(1, D), Wg1, Wg2, bg.reshape(1, D),
      gamma1.reshape(1, D), beta1.reshape(1, D), W1, b1.reshape(1, DFF),
      W2, b2.reshape(1, D), gamma2.reshape(1, D), beta2.reshape(1, D))


def kernel(x, edge_feat, edge_index, Wq, Wk, Wv, We, Wout, bout, Wg, bg,
           gamma1, beta1, W1, b1, W2, b2, gamma2, beta2):
    src = edge_index[0]
    dst = edge_index[1]
    q, kv = _qkv(x, Wq, Wk, Wv)
    eproj = _eproj(edge_feat, We)
    num, den = _edge_phase(q, kv, eproj, src, dst)
    return _epilogue(num, den, x, Wout, bout, Wg, bg, gamma1, beta1,
                     W1, b1, W2, b2, gamma2, beta2)


# speedup vs baseline: 1.5807x; 1.2336x over previous
import functools
import math

import jax
import jax.numpy as jnp
from jax import lax
from jax.experimental import pallas as pl
from jax.experimental.pallas import tpu as pltpu
from jax.experimental.pallas import tpu_sc as plsc

N = 10000
E = 320000
D = 128
H = 8
DH = 16

NC = 2
NS = 16
L = 16
NW = NC * NS
EPW = E // NW
CH = 80
NCHUNK = EPW // CH
ACC_R = ((N + NS * L - 1) // (NS * L)) * NS * L
RPS = ACC_R // NS
ZB = 8
ROW = 136


def _qkv_body(x_ref, wq_ref, wk_ref, wv_ref, q_ref, kv_ref):
    xb = x_ref[...]
    q_ref[...] = jnp.dot(xb, wq_ref[...], preferred_element_type=jnp.float32)
    kv_ref[:, :D] = jnp.dot(xb, wk_ref[...], preferred_element_type=jnp.float32)
    kv_ref[:, D:] = jnp.dot(xb, wv_ref[...], preferred_element_type=jnp.float32)


def _qkv(x, Wq, Wk, Wv):
    blk = 1000
    grid = (N // blk,)
    return pl.pallas_call(
        _qkv_body,
        grid=grid,
        in_specs=[
            pl.BlockSpec((blk, D), lambda i: (i, 0)),
            pl.BlockSpec((D, D), lambda i: (0, 0)),
            pl.BlockSpec((D, D), lambda i: (0, 0)),
            pl.BlockSpec((D, D), lambda i: (0, 0)),
        ],
        out_specs=[
            pl.BlockSpec((blk, D), lambda i: (i, 0)),
            pl.BlockSpec((blk, 2 * D), lambda i: (i, 0)),
        ],
        out_shape=[
            jax.ShapeDtypeStruct((N, D), jnp.float32),
            jax.ShapeDtypeStruct((N, 2 * D), jnp.float32),
        ],
    )(x, Wq, Wk, Wv)


def _eproj_body(ef_ref, we_ref, out_ref):
    out_ref[...] = jnp.dot(ef_ref[...], we_ref[...],
                           preferred_element_type=jnp.float32)


def _eproj(edge_feat, We):
    ef8 = edge_feat.reshape(E // 8, 8 * 16)
    we_bd = jnp.zeros((8 * 16, 8 * H), jnp.float32)
    for j in range(8):
        we_bd = we_bd.at[j * 16:(j + 1) * 16, j * H:(j + 1) * H].set(We)
    blk = 4000
    grid = (E // 8 // blk,)
    out = pl.pallas_call(
        _eproj_body,
        grid=grid,
        in_specs=[
            pl.BlockSpec((blk, 128), lambda i: (i, 0)),
            pl.BlockSpec((128, 64), lambda i: (0, 0)),
        ],
        out_specs=pl.BlockSpec((blk, 64), lambda i: (i, 0)),
        out_shape=jax.ShapeDtypeStruct((E // 8, 64), jnp.float32),
    )(ef8, we_bd)
    return out.reshape(E, H)


def _sc_edge_body(q_hbm, kv_hbm, ep_hbm, src_hbm, dst_hbm, out_hbm,
                  src_v, dst_v, q_v, kv_v, ep_v, msg_v, zb_v, acc,
                  sem1, sem2):
    c = lax.axis_index("c")
    s = lax.axis_index("s")
    wid = s * NC + c
    ebase = wid * EPW

    zero = jnp.zeros((L,), jnp.float32)
    for i in range(ZB):
        for j in range(ROW // L):
            zb_v[i, pl.ds(j * L, L)] = zero
        zb_v[i, pl.ds(ROW - L, L)] = zero

    def _zero_acc(t, carry):
        pltpu.sync_copy(zb_v, acc.at[pl.ds(s * RPS + t * ZB, ZB)])
        return carry
    lax.fori_loop(0, RPS // ZB, _zero_acc, 0)
    plsc.subcore_barrier()

    lanes = lax.iota(jnp.int32, L)

    def _chunk(t, carry):
        base = ebase + t * CH
        pltpu.sync_copy(src_hbm.at[pl.ds(base, CH)], src_v)
        pltpu.sync_copy(dst_hbm.at[pl.ds(base, CH)], dst_v)
        pltpu.sync_copy(ep_hbm.at[pl.ds(base, CH)], ep_v)
        cp1 = pltpu.async_copy(q_hbm.at[dst_v], q_v, sem1)
        cp2 = pltpu.async_copy(kv_hbm.at[src_v], kv_v, sem2)
        cp1.wait()
        cp2.wait()

        def _group(g, carry2):
            rows = g * L + lanes
            for h in range(H):
                accv = jnp.zeros((L,), jnp.float32)
                for i in range(DH):
                    col = jnp.full((L,), h * DH + i, jnp.int32)
                    qc = plsc.load_gather(q_v, [rows, col])
                    kc = plsc.load_gather(kv_v, [rows, col])
                    accv = accv + qc * kc
                epc = plsc.load_gather(ep_v, [rows, jnp.full((L,), h, jnp.int32)])
                alpha = jnp.exp(accv * 0.25 + epc)
                plsc.store_scatter(msg_v, [rows, jnp.full((L,), D + h, jnp.int32)], alpha)
                for i in range(DH):
                    colv = jnp.full((L,), D + h * DH + i, jnp.int32)
                    colm = jnp.full((L,), h * DH + i, jnp.int32)
                    vc = plsc.load_gather(kv_v, [rows, colv])
                    plsc.store_scatter(msg_v, [rows, colm], alpha * vc)
            return carry2
        lax.fori_loop(0, CH // L, _group, 0)
        pltpu.sync_copy(msg_v, acc.at[dst_v], add=True)
        return carry
    lax.fori_loop(0, NCHUNK, _chunk, 0)

    plsc.subcore_barrier()
    pltpu.sync_copy(acc.at[pl.ds(s * RPS, RPS)],
                    out_hbm.at[c, pl.ds(s * RPS, RPS)])


def _edge_phase(q, kv, eproj, src, dst):
    mesh = plsc.VectorSubcoreMesh(core_axis_name="c", subcore_axis_name="s")
    run = pl.kernel(
        _sc_edge_body, mesh=mesh,
        compiler_params=pltpu.CompilerParams(
            needs_layout_passes=False, use_tc_tiling_on_sc=False),
        out_type=jax.ShapeDtypeStruct((NC, ACC_R, ROW), jnp.float32),
        scratch_types=[
            pltpu.VMEM((CH,), jnp.int32),
            pltpu.VMEM((CH,), jnp.int32),
            pltpu.VMEM((CH, D), jnp.float32),
            pltpu.VMEM((CH, 2 * D), jnp.float32),
            pltpu.VMEM((CH, H), jnp.float32),
            pltpu.VMEM((CH, ROW), jnp.float32),
            pltpu.VMEM((ZB, ROW), jnp.float32),
            pltpu.VMEM_SHARED((ACC_R, ROW), jnp.float32),
            pltpu.SemaphoreType.DMA,
            pltpu.SemaphoreType.DMA,
        ],
    )
    return run(q, kv, eproj, src, dst)


def _epi_body(acc_ref, x_ref, wout_ref, bout_ref, wg1_ref, wg2_ref,
              bg_ref, g1_ref, b1n_ref, w1_ref, bb1_ref, w2_ref, bb2_ref,
              g2_ref, b2n_ref, out_ref):
    a = acc_ref[0] + acc_ref[1]
    xb = x_ref[...]
    num = a[:, :D]
    rep = (jnp.arange(ROW)[:, None] == (D + jnp.arange(D)[None, :] // DH)
           ).astype(jnp.float32)
    den_rep = jnp.dot(a, rep, preferred_element_type=jnp.float32)
    agg = num / (den_rep + 1e-20)
    out_lin = jnp.dot(agg, wout_ref[...],
                      preferred_element_type=jnp.float32) + bout_ref[...]
    gz = (jnp.dot(out_lin, wg1_ref[...], preferred_element_type=jnp.float32)
          + jnp.dot(xb, wg2_ref[...], preferred_element_type=jnp.float32)
          + bg_ref[...])
    g = jax.nn.sigmoid(gz)
    h = g * out_lin + (1.0 - g) * xb
    mu = jnp.mean(h, axis=-1, keepdims=True)
    var = jnp.mean((h - mu) ** 2, axis=-1, keepdims=True)
    y = (h - mu) * jax.lax.rsqrt(var + 1e-5) * g1_ref[...] + b1n_ref[...]
    z = jnp.dot(y, w1_ref[...], preferred_element_type=jnp.float32) + bb1_ref[...]
    z = z * jax.nn.sigmoid(z)
    y2 = jnp.dot(z, w2_ref[...], preferred_element_type=jnp.float32) + bb2_ref[...]
    s = y + y2
    mu2 = jnp.mean(s, axis=-1, keepdims=True)
    var2 = jnp.mean((s - mu2) ** 2, axis=-1, keepdims=True)
    out_ref[...] = ((s - mu2) * jax.lax.rsqrt(var2 + 1e-5) * g2_ref[...]
                    + b2n_ref[...])


def _epilogue(acc, x, Wout, bout, Wg, bg, gamma1, beta1, W1, b1, W2, b2,
              gamma2, beta2):
    blk = 1000
    grid = (N // blk,)
    Wg1 = Wg[:D]
    Wg2 = Wg[D:]
    row = lambda i: (i, 0)
    full = lambda r, c: pl.BlockSpec((r, c), lambda i: (0, 0))
    vec = lambda c: pl.BlockSpec((1, c), lambda i: (0, 0))
    DFF = W1.shape[1]
    return pl.pallas_call(
        _epi_body,
        grid=grid,
        in_specs=[
            pl.BlockSpec((NC, blk, ROW), lambda i: (0, i, 0)),
            pl.BlockSpec((blk, D), row),
            full(D, D), vec(D), full(D, D), full(D, D), vec(D),
            vec(D), vec(D), full(D, DFF), vec(DFF), full(DFF, D), vec(D),
            vec(D), vec(D),
        ],
        out_specs=pl.BlockSpec((blk, D), row),
        out_shape=jax.ShapeDtypeStruct((N, D), jnp.float32),
    )(acc, x, Wout, bout.reshape(1, D), Wg1, Wg2, bg.reshape(1, D),
      gamma1.reshape(1, D), beta1.reshape(1, D), W1, b1.reshape(1, DFF),
      W2, b2.reshape(1, D), gamma2.reshape(1, D), beta2.reshape(1, D))


def kernel(x, edge_feat, edge_index, Wq, Wk, Wv, We, Wout, bout, Wg, bg,
           gamma1, beta1, W1, b1, W2, b2, gamma2, beta2):
    src = edge_index[0]
    dst = edge_index[1]
    q, kv = _qkv(x, Wq, Wk, Wv)
    eproj = _eproj(edge_feat, We)
    acc = _edge_phase(q, kv, eproj, src, dst)
    return _epilogue(acc, x, Wout, bout, Wg, bg, gamma1, beta1,
                     W1, b1, W2, b2, gamma2, beta2)


# speedup vs baseline: 3.7465x; 2.3702x over previous
import functools
import math

import jax
import jax.numpy as jnp
from jax import lax
from jax.experimental import pallas as pl
from jax.experimental.pallas import tpu as pltpu
from jax.experimental.pallas import tpu_sc as plsc

N = 10000
E = 320000
D = 128
H = 8
DH = 16

NC = 2
NS = 16
L = 16
NW = NC * NS
EPW = E // NW
CH = 80
NCHUNK = EPW // CH
ACC_R = ((N + NS * L - 1) // (NS * L)) * NS * L
RPS = ACC_R // NS
ZB = 8
ROW = 136


def _qkv_body(x_ref, wq_ref, wk_ref, wv_ref, q_ref, kv_ref):
    xb = x_ref[...]
    q_ref[...] = jnp.dot(xb, wq_ref[...], preferred_element_type=jnp.float32)
    kv_ref[:, :D] = jnp.dot(xb, wk_ref[...], preferred_element_type=jnp.float32)
    kv_ref[:, D:] = jnp.dot(xb, wv_ref[...], preferred_element_type=jnp.float32)


def _qkv(x, Wq, Wk, Wv):
    blk = 1000
    grid = (N // blk,)
    return pl.pallas_call(
        _qkv_body,
        grid=grid,
        in_specs=[
            pl.BlockSpec((blk, D), lambda i: (i, 0)),
            pl.BlockSpec((D, D), lambda i: (0, 0)),
            pl.BlockSpec((D, D), lambda i: (0, 0)),
            pl.BlockSpec((D, D), lambda i: (0, 0)),
        ],
        out_specs=[
            pl.BlockSpec((blk, D), lambda i: (i, 0)),
            pl.BlockSpec((blk, 2 * D), lambda i: (i, 0)),
        ],
        out_shape=[
            jax.ShapeDtypeStruct((N, D), jnp.float32),
            jax.ShapeDtypeStruct((N, 2 * D), jnp.float32),
        ],
    )(x, Wq, Wk, Wv)


def _eproj_body(ef_ref, we_ref, out_ref):
    out_ref[...] = jnp.dot(ef_ref[...], we_ref[...],
                           preferred_element_type=jnp.float32)


def _eproj(edge_feat, We):
    ef8 = edge_feat.reshape(E // 8, 8 * 16)
    we_bd = jnp.zeros((8 * 16, 8 * H), jnp.float32)
    for j in range(8):
        we_bd = we_bd.at[j * 16:(j + 1) * 16, j * H:(j + 1) * H].set(We)
    blk = 4000
    grid = (E // 8 // blk,)
    out = pl.pallas_call(
        _eproj_body,
        grid=grid,
        in_specs=[
            pl.BlockSpec((blk, 128), lambda i: (i, 0)),
            pl.BlockSpec((128, 64), lambda i: (0, 0)),
        ],
        out_specs=pl.BlockSpec((blk, 64), lambda i: (i, 0)),
        out_shape=jax.ShapeDtypeStruct((E // 8, 64), jnp.float32),
    )(ef8, we_bd)
    return out.reshape(E, H)


def _sc_edge_body(q_hbm, kv_hbm, ep_hbm, src_hbm, dst_hbm, out_hbm,
                  src_v, dst_v, q_v, kv_v, ep_v, msg_v, zb_v, acc,
                  sem1, sem2):
    c = lax.axis_index("c")
    s = lax.axis_index("s")
    wid = s * NC + c
    ebase = wid * EPW

    zero = jnp.zeros((L,), jnp.float32)
    for i in range(ZB):
        for j in range(ROW // L):
            zb_v[i, pl.ds(j * L, L)] = zero
        zb_v[i, pl.ds(ROW - L, L)] = zero

    def _zero_acc(t, carry):
        pltpu.sync_copy(zb_v, acc.at[pl.ds(s * RPS + t * ZB, ZB)])
        return carry
    lax.fori_loop(0, RPS // ZB, _zero_acc, 0)
    plsc.subcore_barrier()

    lanes = lax.iota(jnp.int32, L)

    def _chunk(t, carry):
        base = ebase + t * CH
        pltpu.sync_copy(src_hbm.at[pl.ds(base, CH)], src_v)
        pltpu.sync_copy(dst_hbm.at[pl.ds(base, CH)], dst_v)
        pltpu.sync_copy(
            ep_hbm.at[pl.ds(wid * (EPW // 2) + t * (CH // 2), CH // 2)], ep_v)
        cp1 = pltpu.async_copy(q_hbm.at[dst_v], q_v, sem1)
        cp2 = pltpu.async_copy(kv_hbm.at[src_v], kv_v, sem2)
        cp1.wait()
        cp2.wait()

        def _pair(j, carry2):
            e0 = 2 * j
            score = jnp.zeros((L,), jnp.float32)
            for off, e in ((0, e0), (8, e0 + 1)):
                for h in range(H):
                    qh = q_v[e, pl.ds(h * DH, DH)]
                    kh = kv_v[e, pl.ds(h * DH, DH)]
                    dot = jnp.sum(qh * kh)
                    score = jnp.where(lanes == off + h, dot, score)
            alpha16 = jnp.exp(score * 0.25 + ep_v[j, :])
            drows = e0 + lax.shift_right_logical(lanes, 2 + 1)
            dcols = D + (lanes & 7)
            plsc.store_scatter(msg_v, [drows, dcols], alpha16)
            for off, e in ((0, e0), (8, e0 + 1)):
                for h in range(H):
                    a = jnp.sum(jnp.where(lanes == off + h, alpha16, 0.0))
                    vh = kv_v[e, pl.ds(D + h * DH, DH)]
                    msg_v[e, pl.ds(h * DH, DH)] = a * vh
            return carry2
        lax.fori_loop(0, CH // 2, _pair, 0)
        pltpu.sync_copy(msg_v, acc.at[dst_v], add=True)
        return carry
    lax.fori_loop(0, NCHUNK, _chunk, 0)

    plsc.subcore_barrier()
    pltpu.sync_copy(acc.at[pl.ds(s * RPS, RPS)],
                    out_hbm.at[c, pl.ds(s * RPS, RPS)])


def _edge_phase(q, kv, eproj, src, dst):
    mesh = plsc.VectorSubcoreMesh(core_axis_name="c", subcore_axis_name="s")
    run = pl.kernel(
        _sc_edge_body, mesh=mesh,
        compiler_params=pltpu.CompilerParams(
            needs_layout_passes=False, use_tc_tiling_on_sc=False),
        out_type=jax.ShapeDtypeStruct((NC, ACC_R, ROW), jnp.float32),
        scratch_types=[
            pltpu.VMEM((CH,), jnp.int32),
            pltpu.VMEM((CH,), jnp.int32),
            pltpu.VMEM((CH, D), jnp.float32),
            pltpu.VMEM((CH, 2 * D), jnp.float32),
            pltpu.VMEM((CH // 2, 2 * H), jnp.float32),
            pltpu.VMEM((CH, ROW), jnp.float32),
            pltpu.VMEM((ZB, ROW), jnp.float32),
            pltpu.VMEM_SHARED((ACC_R, ROW), jnp.float32),
            pltpu.SemaphoreType.DMA,
            pltpu.SemaphoreType.DMA,
        ],
    )
    return run(q, kv, eproj.reshape(E // 2, 2 * H), src, dst)


def _epi_body(acc_ref, x_ref, wout_ref, bout_ref, wg1_ref, wg2_ref,
              bg_ref, g1_ref, b1n_ref, w1_ref, bb1_ref, w2_ref, bb2_ref,
              g2_ref, b2n_ref, out_ref):
    a = acc_ref[0] + acc_ref[1]
    xb = x_ref[...]
    num = a[:, :D]
    rep = (jnp.arange(ROW)[:, None] == (D + jnp.arange(D)[None, :] // DH)
           ).astype(jnp.float32)
    den_rep = jnp.dot(a, rep, preferred_element_type=jnp.float32)
    agg = num / (den_rep + 1e-20)
    out_lin = jnp.dot(agg, wout_ref[...],
                      preferred_element_type=jnp.float32) + bout_ref[...]
    gz = (jnp.dot(out_lin, wg1_ref[...], preferred_element_type=jnp.float32)
          + jnp.dot(xb, wg2_ref[...], preferred_element_type=jnp.float32)
          + bg_ref[...])
    g = jax.nn.sigmoid(gz)
    h = g * out_lin + (1.0 - g) * xb
    mu = jnp.mean(h, axis=-1, keepdims=True)
    var = jnp.mean((h - mu) ** 2, axis=-1, keepdims=True)
    y = (h - mu) * jax.lax.rsqrt(var + 1e-5) * g1_ref[...] + b1n_ref[...]
    z = jnp.dot(y, w1_ref[...], preferred_element_type=jnp.float32) + bb1_ref[...]
    z = z * jax.nn.sigmoid(z)
    y2 = jnp.dot(z, w2_ref[...], preferred_element_type=jnp.float32) + bb2_ref[...]
    s = y + y2
    mu2 = jnp.mean(s, axis=-1, keepdims=True)
    var2 = jnp.mean((s - mu2) ** 2, axis=-1, keepdims=True)
    out_ref[...] = ((s - mu2) * jax.lax.rsqrt(var2 + 1e-5) * g2_ref[...]
                    + b2n_ref[...])


def _epilogue(acc, x, Wout, bout, Wg, bg, gamma1, beta1, W1, b1, W2, b2,
              gamma2, beta2):
    blk = 1000
    grid = (N // blk,)
    Wg1 = Wg[:D]
    Wg2 = Wg[D:]
    row = lambda i: (i, 0)
    full = lambda r, c: pl.BlockSpec((r, c), lambda i: (0, 0))
    vec = lambda c: pl.BlockSpec((1, c), lambda i: (0, 0))
    DFF = W1.shape[1]
    return pl.pallas_call(
        _epi_body,
        grid=grid,
        in_specs=[
            pl.BlockSpec((NC, blk, ROW), lambda i: (0, i, 0)),
            pl.BlockSpec((blk, D), row),
            full(D, D), vec(D), full(D, D), full(D, D), vec(D),
            vec(D), vec(D), full(D, DFF), vec(DFF), full(DFF, D), vec(D),
            vec(D), vec(D),
        ],
        out_specs=pl.BlockSpec((blk, D), row),
        out_shape=jax.ShapeDtypeStruct((N, D), jnp.float32),
    )(acc, x, Wout, bout.reshape(1, D), Wg1, Wg2, bg.reshape(1, D),
      gamma1.reshape(1, D), beta1.reshape(1, D), W1, b1.reshape(1, DFF),
      W2, b2.reshape(1, D), gamma2.reshape(1, D), beta2.reshape(1, D))


def kernel(x, edge_feat, edge_index, Wq, Wk, Wv, We, Wout, bout, Wg, bg,
           gamma1, beta1, W1, b1, W2, b2, gamma2, beta2):
    src = edge_index[0]
    dst = edge_index[1]
    q, kv = _qkv(x, Wq, Wk, Wv)
    eproj = _eproj(edge_feat, We)
    acc = _edge_phase(q, kv, eproj, src, dst)
    return _epilogue(acc, x, Wout, bout, Wg, bg, gamma1, beta1,
                     W1, b1, W2, b2, gamma2, beta2)


# speedup vs baseline: 3.8108x; 1.0171x over previous
import functools
import math

import jax
import jax.numpy as jnp
from jax import lax
from jax.experimental import pallas as pl
from jax.experimental.pallas import tpu as pltpu
from jax.experimental.pallas import tpu_sc as plsc

N = 10000
E = 320000
D = 128
H = 8
DH = 16

NC = 2
NS = 16
L = 16
NW = NC * NS
EPW = E // NW
CH = 80
NCHUNK = EPW // CH
ACC_R = ((N + NS * L - 1) // (NS * L)) * NS * L
RPS = ACC_R // NS
ZB = 4
ROW = 136


def _qkv_body(x_ref, wq_ref, wk_ref, wv_ref, q_ref, kv_ref):
    xb = x_ref[...]
    q_ref[...] = jnp.dot(xb, wq_ref[...], preferred_element_type=jnp.float32)
    kv_ref[:, :D] = jnp.dot(xb, wk_ref[...], preferred_element_type=jnp.float32)
    kv_ref[:, D:] = jnp.dot(xb, wv_ref[...], preferred_element_type=jnp.float32)


def _qkv(x, Wq, Wk, Wv):
    blk = 1000
    grid = (N // blk,)
    return pl.pallas_call(
        _qkv_body,
        grid=grid,
        in_specs=[
            pl.BlockSpec((blk, D), lambda i: (i, 0)),
            pl.BlockSpec((D, D), lambda i: (0, 0)),
            pl.BlockSpec((D, D), lambda i: (0, 0)),
            pl.BlockSpec((D, D), lambda i: (0, 0)),
        ],
        out_specs=[
            pl.BlockSpec((blk, D), lambda i: (i, 0)),
            pl.BlockSpec((blk, 2 * D), lambda i: (i, 0)),
        ],
        out_shape=[
            jax.ShapeDtypeStruct((N, D), jnp.float32),
            jax.ShapeDtypeStruct((N, 2 * D), jnp.float32),
        ],
    )(x, Wq, Wk, Wv)


def _eproj_body(ef_ref, we_ref, out_ref):
    out_ref[...] = jnp.dot(ef_ref[...], we_ref[...],
                           preferred_element_type=jnp.float32)


def _eproj(edge_feat, We):
    ef8 = edge_feat.reshape(E // 8, 8 * 16)
    we_bd = jnp.zeros((8 * 16, 8 * H), jnp.float32)
    for j in range(8):
        we_bd = we_bd.at[j * 16:(j + 1) * 16, j * H:(j + 1) * H].set(We)
    blk = 4000
    grid = (E // 8 // blk,)
    out = pl.pallas_call(
        _eproj_body,
        grid=grid,
        in_specs=[
            pl.BlockSpec((blk, 128), lambda i: (i, 0)),
            pl.BlockSpec((128, 64), lambda i: (0, 0)),
        ],
        out_specs=pl.BlockSpec((blk, 64), lambda i: (i, 0)),
        out_shape=jax.ShapeDtypeStruct((E // 8, 64), jnp.float32),
    )(ef8, we_bd)
    return out.reshape(E, H)


def _sc_edge_body(q_hbm, kv_hbm, ep_hbm, src_hbm, dst_hbm, out_hbm,
                  src_v, dst_v, q_v, kv_v, ep_v, msg_v, zb_v, acc,
                  sem1, sem2):
    c = lax.axis_index("c")
    s = lax.axis_index("s")
    wid = s * NC + c
    ebase = wid * EPW

    zero = jnp.zeros((L,), jnp.float32)
    for i in range(ZB):
        for j in range(ROW // L):
            zb_v[i, pl.ds(j * L, L)] = zero
        zb_v[i, pl.ds(ROW - L, L)] = zero

    def _zero_acc(t, carry):
        pltpu.sync_copy(zb_v, acc.at[pl.ds(s * RPS + t * ZB, ZB)])
        return carry
    lax.fori_loop(0, RPS // ZB, _zero_acc, 0)
    plsc.subcore_barrier()

    lanes = lax.iota(jnp.int32, L)

    def _chunk(t, carry):
        base = ebase + t * CH
        pltpu.sync_copy(src_hbm.at[pl.ds(base, CH)], src_v)
        pltpu.sync_copy(dst_hbm.at[pl.ds(base, CH)], dst_v)
        pltpu.sync_copy(
            ep_hbm.at[pl.ds(wid * (EPW // 2) + t * (CH // 2), CH // 2)], ep_v)
        cp1 = pltpu.async_copy(q_hbm.at[dst_v], q_v, sem1)
        cp2 = pltpu.async_copy(kv_hbm.at[src_v], kv_v, sem2)
        cp1.wait()
        cp2.wait()

        def _pair(j, carry2):
            e0 = 2 * j
            score = jnp.zeros((L,), jnp.float32)
            for off, e in ((0, e0), (8, e0 + 1)):
                for h in range(H):
                    qh = q_v[e, pl.ds(h * DH, DH)]
                    kh = kv_v[e, pl.ds(h * DH, DH)]
                    score = jnp.where(lanes == off + h, jnp.sum(qh * kh),
                                      score)
            alpha16 = jnp.exp(score * 0.25 + ep_v[j, :])
            drows = e0 + lax.shift_right_logical(lanes, 2 + 1)
            dcols = D + (lanes & 7)
            plsc.store_scatter(msg_v, [drows, dcols], alpha16)
            for off, e in ((0, e0), (8, e0 + 1)):
                for h in range(H):
                    a = alpha16[off + h]
                    vh = kv_v[e, pl.ds(D + h * DH, DH)]
                    msg_v[e, pl.ds(h * DH, DH)] = a * vh
            return carry2
        lax.fori_loop(0, CH // 2, _pair, 0)
        pltpu.sync_copy(msg_v, acc.at[dst_v], add=True)
        return carry
    lax.fori_loop(0, NCHUNK, _chunk, 0)

    plsc.subcore_barrier()
    pltpu.sync_copy(acc.at[pl.ds(s * RPS, RPS)],
                    out_hbm.at[c, pl.ds(s * RPS, RPS)])


def _edge_phase(q, kv, eproj, src, dst):
    mesh = plsc.VectorSubcoreMesh(core_axis_name="c", subcore_axis_name="s")
    run = pl.kernel(
        _sc_edge_body, mesh=mesh,
        compiler_params=pltpu.CompilerParams(
            needs_layout_passes=False, use_tc_tiling_on_sc=False),
        out_type=jax.ShapeDtypeStruct((NC, ACC_R, ROW), jnp.float32),
        scratch_types=[
            pltpu.VMEM((CH,), jnp.int32),
            pltpu.VMEM((CH,), jnp.int32),
            pltpu.VMEM((CH, D), jnp.float32),
            pltpu.VMEM((CH, 2 * D), jnp.float32),
            pltpu.VMEM((CH // 2, 2 * H), jnp.float32),
            pltpu.VMEM((CH, ROW), jnp.float32),
            pltpu.VMEM((ZB, ROW), jnp.float32),
            pltpu.VMEM_SHARED((ACC_R, ROW), jnp.float32),
            pltpu.SemaphoreType.DMA,
            pltpu.SemaphoreType.DMA,
        ],
    )
    return run(q, kv, eproj.reshape(E // 2, 2 * H), src, dst)


def _epi_body(acc_ref, x_ref, wout_ref, bout_ref, wg1_ref, wg2_ref,
              bg_ref, g1_ref, b1n_ref, w1_ref, bb1_ref, w2_ref, bb2_ref,
              g2_ref, b2n_ref, out_ref):
    a = acc_ref[0] + acc_ref[1]
    xb = x_ref[...]
    num = a[:, :D]
    rep = (jnp.arange(ROW)[:, None] == (D + jnp.arange(D)[None, :] // DH)
           ).astype(jnp.float32)
    den_rep = jnp.dot(a, rep, preferred_element_type=jnp.float32)
    agg = num / (den_rep + 1e-20)
    out_lin = jnp.dot(agg, wout_ref[...],
                      preferred_element_type=jnp.float32) + bout_ref[...]
    gz = (jnp.dot(out_lin, wg1_ref[...], preferred_element_type=jnp.float32)
          + jnp.dot(xb, wg2_ref[...], preferred_element_type=jnp.float32)
          + bg_ref[...])
    g = jax.nn.sigmoid(gz)
    h = g * out_lin + (1.0 - g) * xb
    mu = jnp.mean(h, axis=-1, keepdims=True)
    var = jnp.mean((h - mu) ** 2, axis=-1, keepdims=True)
    y = (h - mu) * jax.lax.rsqrt(var + 1e-5) * g1_ref[...] + b1n_ref[...]
    z = jnp.dot(y, w1_ref[...], preferred_element_type=jnp.float32) + bb1_ref[...]
    z = z * jax.nn.sigmoid(z)
    y2 = jnp.dot(z, w2_ref[...], preferred_element_type=jnp.float32) + bb2_ref[...]
    s = y + y2
    mu2 = jnp.mean(s, axis=-1, keepdims=True)
    var2 = jnp.mean((s - mu2) ** 2, axis=-1, keepdims=True)
    out_ref[...] = ((s - mu2) * jax.lax.rsqrt(var2 + 1e-5) * g2_ref[...]
                    + b2n_ref[...])


def _epilogue(acc, x, Wout, bout, Wg, bg, gamma1, beta1, W1, b1, W2, b2,
              gamma2, beta2):
    blk = 1000
    grid = (N // blk,)
    Wg1 = Wg[:D]
    Wg2 = Wg[D:]
    row = lambda i: (i, 0)
    full = lambda r, c: pl.BlockSpec((r, c), lambda i: (0, 0))
    vec = lambda c: pl.BlockSpec((1, c), lambda i: (0, 0))
    DFF = W1.shape[1]
    return pl.pallas_call(
        _epi_body,
        grid=grid,
        in_specs=[
            pl.BlockSpec((NC, blk, ROW), lambda i: (0, i, 0)),
            pl.BlockSpec((blk, D), row),
            full(D, D), vec(D), full(D, D), full(D, D), vec(D),
            vec(D), vec(D), full(D, DFF), vec(DFF), full(DFF, D), vec(D),
            vec(D), vec(D),
        ],
        out_specs=pl.BlockSpec((blk, D), row),
        out_shape=jax.ShapeDtypeStruct((N, D), jnp.float32),
    )(acc, x, Wout, bout.reshape(1, D), Wg1, Wg2, bg.reshape(1, D),
      gamma1.reshape(1, D), beta1.reshape(1, D), W1, b1.reshape(1, DFF),
      W2, b2.reshape(1, D), gamma2.reshape(1, D), beta2.reshape(1, D))


def kernel(x, edge_feat, edge_index, Wq, Wk, Wv, We, Wout, bout, Wg, bg,
           gamma1, beta1, W1, b1, W2, b2, gamma2, beta2):
    src = edge_index[0]
    dst = edge_index[1]
    q, kv = _qkv(x, Wq, Wk, Wv)
    eproj = _eproj(edge_feat, We)
    acc = _edge_phase(q, kv, eproj, src, dst)
    return _epilogue(acc, x, Wout, bout, Wg, bg, gamma1, beta1,
                     W1, b1, W2, b2, gamma2, beta2)


# speedup vs baseline: 4.2044x; 1.1033x over previous
import functools
import math

import jax
import jax.numpy as jnp
from jax import lax
from jax.experimental import pallas as pl
from jax.experimental.pallas import tpu as pltpu
from jax.experimental.pallas import tpu_sc as plsc

N = 10000
E = 320000
D = 128
H = 8
DH = 16

NC = 2
NS = 16
L = 16
NW = NC * NS
EPW = E // NW
CH = 40
NCHUNK = EPW // CH
ACC_R = ((N + NS * L - 1) // (NS * L)) * NS * L
RPS = ACC_R // NS
ZB = 4
ROW = 136


def _qkv_body(x_ref, wq_ref, wk_ref, wv_ref, q_ref, kv_ref):
    xb = x_ref[...]
    q_ref[...] = jnp.dot(xb, wq_ref[...], preferred_element_type=jnp.float32)
    kv_ref[:, :D] = jnp.dot(xb, wk_ref[...], preferred_element_type=jnp.float32)
    kv_ref[:, D:] = jnp.dot(xb, wv_ref[...], preferred_element_type=jnp.float32)


def _qkv(x, Wq, Wk, Wv):
    blk = 1000
    grid = (N // blk,)
    return pl.pallas_call(
        _qkv_body,
        grid=grid,
        in_specs=[
            pl.BlockSpec((blk, D), lambda i: (i, 0)),
            pl.BlockSpec((D, D), lambda i: (0, 0)),
            pl.BlockSpec((D, D), lambda i: (0, 0)),
            pl.BlockSpec((D, D), lambda i: (0, 0)),
        ],
        out_specs=[
            pl.BlockSpec((blk, D), lambda i: (i, 0)),
            pl.BlockSpec((blk, 2 * D), lambda i: (i, 0)),
        ],
        out_shape=[
            jax.ShapeDtypeStruct((N, D), jnp.float32),
            jax.ShapeDtypeStruct((N, 2 * D), jnp.float32),
        ],
    )(x, Wq, Wk, Wv)


def _eproj_body(ef_ref, we_ref, out_ref):
    out_ref[...] = jnp.dot(ef_ref[...], we_ref[...],
                           preferred_element_type=jnp.float32)


def _eproj(edge_feat, We):
    ef8 = edge_feat.reshape(E // 8, 8 * 16)
    we_bd = jnp.zeros((8 * 16, 8 * H), jnp.float32)
    for j in range(8):
        we_bd = we_bd.at[j * 16:(j + 1) * 16, j * H:(j + 1) * H].set(We)
    blk = 4000
    grid = (E // 8 // blk,)
    out = pl.pallas_call(
        _eproj_body,
        grid=grid,
        in_specs=[
            pl.BlockSpec((blk, 128), lambda i: (i, 0)),
            pl.BlockSpec((128, 64), lambda i: (0, 0)),
        ],
        out_specs=pl.BlockSpec((blk, 64), lambda i: (i, 0)),
        out_shape=jax.ShapeDtypeStruct((E // 8, 64), jnp.float32),
    )(ef8, we_bd)
    return out.reshape(E, H)


def _sc_edge_body(q_hbm, kv_hbm, ep_hbm, src_hbm, dst_hbm, out_hbm,
                  s0, s1, d0, d1, q0, q1, kv0, kv1, e0v, e1v, m0, m1,
                  zb_v, acc, sq0, sq1, sk0, sk1, ss0, ss1):
    c = lax.axis_index("c")
    s = lax.axis_index("s")
    wid = s * NC + c
    ebase = wid * EPW

    srcs = (s0, s1)
    dsts = (d0, d1)
    qs = (q0, q1)
    kvs = (kv0, kv1)
    eps = (e0v, e1v)
    msgs = (m0, m1)
    semq = (sq0, sq1)
    semk = (sk0, sk1)
    semsc = (ss0, ss1)

    zero = jnp.zeros((L,), jnp.float32)
    for i in range(ZB):
        for j in range(ROW // L):
            zb_v[i, pl.ds(j * L, L)] = zero
        zb_v[i, pl.ds(ROW - L, L)] = zero

    def _zero_acc(t, carry):
        pltpu.sync_copy(zb_v, acc.at[pl.ds(s * RPS + t * ZB, ZB)])
        return carry
    lax.fori_loop(0, RPS // ZB, _zero_acc, 0)
    plsc.subcore_barrier()

    lanes = lax.iota(jnp.int32, L)

    def _fetch(t, b):
        base = ebase + t * CH
        pltpu.sync_copy(src_hbm.at[pl.ds(base, CH)], srcs[b])
        pltpu.sync_copy(dst_hbm.at[pl.ds(base, CH)], dsts[b])
        pltpu.sync_copy(
            ep_hbm.at[pl.ds(wid * (EPW // 2) + t * (CH // 2), CH // 2)],
            eps[b])
        pltpu.async_copy(q_hbm.at[dsts[b]], qs[b], semq[b])
        pltpu.async_copy(kv_hbm.at[srcs[b]], kvs[b], semk[b])

    def _compute(b):
        q_v, kv_v, ep_v, msg_v = qs[b], kvs[b], eps[b], msgs[b]

        def _pair(j, carry2):
            e0 = 2 * j
            score = jnp.zeros((L,), jnp.float32)
            for off, e in ((0, e0), (8, e0 + 1)):
                for h in range(H):
                    qh = q_v[e, pl.ds(h * DH, DH)]
                    kh = kv_v[e, pl.ds(h * DH, DH)]
                    score = jnp.where(lanes == off + h, jnp.sum(qh * kh),
                                      score)
            alpha16 = jnp.exp(score * 0.25 + ep_v[j, :])
            drows = e0 + lax.shift_right_logical(lanes, 2 + 1)
            dcols = D + (lanes & 7)
            plsc.store_scatter(msg_v, [drows, dcols], alpha16)
            for off, e in ((0, e0), (8, e0 + 1)):
                for h in range(H):
                    a = alpha16[off + h]
                    vh = kv_v[e, pl.ds(D + h * DH, DH)]
                    msg_v[e, pl.ds(h * DH, DH)] = a * vh
            return carry2
        lax.fori_loop(0, CH // 2, _pair, 0)

    _fetch(0, 0)

    def _step(u, carry):
        for b in range(2):
            t = 2 * u + b
            pltpu.make_async_copy(q_hbm.at[dsts[b]], qs[b], semq[b]).wait()
            pltpu.make_async_copy(kv_hbm.at[srcs[b]], kvs[b], semk[b]).wait()

            @pl.when(t + 1 < NCHUNK)
            def _():
                _fetch(t + 1, 1 - b)

            @pl.when(t >= 2)
            def _():
                pltpu.make_async_copy(
                    msgs[b], acc.at[dsts[b]], semsc[b]).wait()

            _compute(b)
            pltpu.async_copy(msgs[b], acc.at[dsts[b]], semsc[b], add=True)
        return carry
    lax.fori_loop(0, NCHUNK // 2, _step, 0)

    for b in range(2):
        pltpu.make_async_copy(msgs[b], acc.at[dsts[b]], semsc[b]).wait()
    plsc.subcore_barrier()
    pltpu.sync_copy(acc.at[pl.ds(s * RPS, RPS)],
                    out_hbm.at[c, pl.ds(s * RPS, RPS)])


def _edge_phase(q, kv, eproj, src, dst):
    mesh = plsc.VectorSubcoreMesh(core_axis_name="c", subcore_axis_name="s")
    run = pl.kernel(
        _sc_edge_body, mesh=mesh,
        compiler_params=pltpu.CompilerParams(
            needs_layout_passes=False, use_tc_tiling_on_sc=False),
        out_type=jax.ShapeDtypeStruct((NC, ACC_R, ROW), jnp.float32),
        scratch_types=(
            [pltpu.VMEM((CH,), jnp.int32)] * 4
            + [pltpu.VMEM((CH, D), jnp.float32)] * 2
            + [pltpu.VMEM((CH, 2 * D), jnp.float32)] * 2
            + [pltpu.VMEM((CH // 2, 2 * H), jnp.float32)] * 2
            + [pltpu.VMEM((CH, ROW), jnp.float32)] * 2
            + [pltpu.VMEM((ZB, ROW), jnp.float32)]
            + [pltpu.VMEM_SHARED((ACC_R, ROW), jnp.float32)]
            + [pltpu.SemaphoreType.DMA] * 6
        ),
    )
    return run(q, kv, eproj.reshape(E // 2, 2 * H), src, dst)


def _epi_body(acc_ref, x_ref, wout_ref, bout_ref, wg1_ref, wg2_ref,
              bg_ref, g1_ref, b1n_ref, w1_ref, bb1_ref, w2_ref, bb2_ref,
              g2_ref, b2n_ref, out_ref):
    a = acc_ref[0] + acc_ref[1]
    xb = x_ref[...]
    num = a[:, :D]
    rep = (jnp.arange(ROW)[:, None] == (D + jnp.arange(D)[None, :] // DH)
           ).astype(jnp.float32)
    den_rep = jnp.dot(a, rep, preferred_element_type=jnp.float32)
    agg = num / (den_rep + 1e-20)
    out_lin = jnp.dot(agg, wout_ref[...],
                      preferred_element_type=jnp.float32) + bout_ref[...]
    gz = (jnp.dot(out_lin, wg1_ref[...], preferred_element_type=jnp.float32)
          + jnp.dot(xb, wg2_ref[...], preferred_element_type=jnp.float32)
          + bg_ref[...])
    g = jax.nn.sigmoid(gz)
    h = g * out_lin + (1.0 - g) * xb
    mu = jnp.mean(h, axis=-1, keepdims=True)
    var = jnp.mean((h - mu) ** 2, axis=-1, keepdims=True)
    y = (h - mu) * jax.lax.rsqrt(var + 1e-5) * g1_ref[...] + b1n_ref[...]
    z = jnp.dot(y, w1_ref[...], preferred_element_type=jnp.float32) + bb1_ref[...]
    z = z * jax.nn.sigmoid(z)
    y2 = jnp.dot(z, w2_ref[...], preferred_element_type=jnp.float32) + bb2_ref[...]
    s = y + y2
    mu2 = jnp.mean(s, axis=-1, keepdims=True)
    var2 = jnp.mean((s - mu2) ** 2, axis=-1, keepdims=True)
    out_ref[...] = ((s - mu2) * jax.lax.rsqrt(var2 + 1e-5) * g2_ref[...]
                    + b2n_ref[...])


def _epilogue(acc, x, Wout, bout, Wg, bg, gamma1, beta1, W1, b1, W2, b2,
              gamma2, beta2):
    blk = 1000
    grid = (N // blk,)
    Wg1 = Wg[:D]
    Wg2 = Wg[D:]
    row = lambda i: (i, 0)
    full = lambda r, c: pl.BlockSpec((r, c), lambda i: (0, 0))
    vec = lambda c: pl.BlockSpec((1, c), lambda i: (0, 0))
    DFF = W1.shape[1]
    return pl.pallas_call(
        _epi_body,
        grid=grid,
        in_specs=[
            pl.BlockSpec((NC, blk, ROW), lambda i: (0, i, 0)),
            pl.BlockSpec((blk, D), row),
            full(D, D), vec(D), full(D, D), full(D, D), vec(D),
            vec(D), vec(D), full(D, DFF), vec(DFF), full(DFF, D), vec(D),
            vec(D), vec(D),
        ],
        out_specs=pl.BlockSpec((blk, D), row),
        out_shape=jax.ShapeDtypeStruct((N, D), jnp.float32),
    )(acc, x, Wout, bout.reshape(1, D), Wg1, Wg2, bg.reshape(1, D),
      gamma1.reshape(1, D), beta1.reshape(1, D), W1, b1.reshape(1, DFF),
      W2, b2.reshape(1, D), gamma2.reshape(1, D), beta2.reshape(1, D))


def kernel(x, edge_feat, edge_index, Wq, Wk, Wv, We, Wout, bout, Wg, bg,
           gamma1, beta1, W1, b1, W2, b2, gamma2, beta2):
    src = edge_index[0]
    dst = edge_index[1]
    q, kv = _qkv(x, Wq, Wk, Wv)
    eproj = _eproj(edge_feat, We)
    acc = _edge_phase(q, kv, eproj, src, dst)
    return _epilogue(acc, x, Wout, bout, Wg, bg, gamma1, beta1,
                     W1, b1, W2, b2, gamma2, beta2)


# speedup vs baseline: 5.1280x; 1.2197x over previous
import functools
import math

import jax
import jax.numpy as jnp
from jax import lax
from jax.experimental import pallas as pl
from jax.experimental.pallas import tpu as pltpu
from jax.experimental.pallas import tpu_sc as plsc

N = 10000
E = 320000
D = 128
H = 8
DH = 16

NC = 2
NS = 16
L = 16
NW = NC * NS
EPW = E // NW
CH = 40
NCHUNK = EPW // CH
ACC_R = ((N + NS * L - 1) // (NS * L)) * NS * L
RPS = ACC_R // NS
ZB = 4
ROW = 136


def _qkv_body(x_ref, wq_ref, wk_ref, wv_ref, q_ref, kv_ref):
    xb = x_ref[...]
    q_ref[...] = jnp.dot(xb, wq_ref[...], preferred_element_type=jnp.float32)
    kv_ref[:, :D] = jnp.dot(xb, wk_ref[...], preferred_element_type=jnp.float32)
    kv_ref[:, D:] = jnp.dot(xb, wv_ref[...], preferred_element_type=jnp.float32)


def _qkv(x, Wq, Wk, Wv):
    blk = 1000
    grid = (N // blk,)
    return pl.pallas_call(
        _qkv_body,
        grid=grid,
        in_specs=[
            pl.BlockSpec((blk, D), lambda i: (i, 0)),
            pl.BlockSpec((D, D), lambda i: (0, 0)),
            pl.BlockSpec((D, D), lambda i: (0, 0)),
            pl.BlockSpec((D, D), lambda i: (0, 0)),
        ],
        out_specs=[
            pl.BlockSpec((blk, D), lambda i: (i, 0)),
            pl.BlockSpec((blk, 2 * D), lambda i: (i, 0)),
        ],
        out_shape=[
            jax.ShapeDtypeStruct((N, D), jnp.float32),
            jax.ShapeDtypeStruct((N, 2 * D), jnp.float32),
        ],
    )(x, Wq, Wk, Wv)


def _eproj_body(ef_ref, we_ref, out_ref):
    out_ref[...] = jnp.dot(ef_ref[...], we_ref[...],
                           preferred_element_type=jnp.float32)


def _eproj(edge_feat, We):
    ef8 = edge_feat.reshape(E // 8, 8 * 16)
    we_bd = jnp.zeros((8 * 16, 8 * H), jnp.float32)
    for j in range(8):
        we_bd = we_bd.at[j * 16:(j + 1) * 16, j * H:(j + 1) * H].set(We)
    blk = 4000
    grid = (E // 8 // blk,)
    out = pl.pallas_call(
        _eproj_body,
        grid=grid,
        in_specs=[
            pl.BlockSpec((blk, 128), lambda i: (i, 0)),
            pl.BlockSpec((128, 64), lambda i: (0, 0)),
        ],
        out_specs=pl.BlockSpec((blk, 64), lambda i: (i, 0)),
        out_shape=jax.ShapeDtypeStruct((E // 8, 64), jnp.float32),
    )(ef8, we_bd)
    return out.reshape(E, H)


def _sc_edge_body(q_hbm, kv_hbm, ep_hbm, src_hbm, dst_hbm, out_hbm,
                  s0, s1, d0, d1, dc0, dc1, q0, q1, kv0, kv1, e0v, e1v,
                  m0, m1, zb_v, acc, sq0, sq1, sk0, sk1, ss0, ss1,
                  si0, si1):
    c = lax.axis_index("c")
    s = lax.axis_index("s")
    wid = s * NC + c
    ebase = wid * EPW

    srcs = (s0, s1)
    dsts = (d0, d1)
    dscat = (dc0, dc1)
    qs = (q0, q1)
    kvs = (kv0, kv1)
    eps = (e0v, e1v)
    msgs = (m0, m1)
    semq = (sq0, sq1)
    semk = (sk0, sk1)
    semsc = (ss0, ss1)
    semi = (si0, si1)

    zero = jnp.zeros((L,), jnp.float32)
    for i in range(ZB):
        for j in range(ROW // L):
            zb_v[i, pl.ds(j * L, L)] = zero
        zb_v[i, pl.ds(ROW - L, L)] = zero

    def _zero_acc(t, carry):
        pltpu.sync_copy(zb_v, acc.at[pl.ds(s * RPS + t * ZB, ZB)])
        return carry
    lax.fori_loop(0, RPS // ZB, _zero_acc, 0)
    plsc.subcore_barrier()

    lanes = lax.iota(jnp.int32, L)

    def _fetch_idx(t, b):
        base = ebase + t * CH
        pltpu.async_copy(src_hbm.at[pl.ds(base, CH)], srcs[b], semi[b])
        pltpu.async_copy(dst_hbm.at[pl.ds(base, CH)], dsts[b], semi[b])
        pltpu.async_copy(
            ep_hbm.at[pl.ds(wid * (EPW // 2) + t * (CH // 2), CH // 2)],
            eps[b], semi[b])

    def _wait_idx(b):
        pltpu.make_async_copy(src_hbm.at[pl.ds(0, CH)], srcs[b],
                              semi[b]).wait()
        pltpu.make_async_copy(dst_hbm.at[pl.ds(0, CH)], dsts[b],
                              semi[b]).wait()
        pltpu.make_async_copy(ep_hbm.at[pl.ds(0, CH // 2)], eps[b],
                              semi[b]).wait()

    def _start_gather(b):
        pltpu.async_copy(q_hbm.at[dsts[b]], qs[b], semq[b])
        pltpu.async_copy(kv_hbm.at[srcs[b]], kvs[b], semk[b])

    def _compute(b):
        q_v, kv_v, ep_v, msg_v = qs[b], kvs[b], eps[b], msgs[b]

        def _pair(j, carry2):
            e0 = 2 * j
            score = jnp.zeros((L,), jnp.float32)
            for off, e in ((0, e0), (8, e0 + 1)):
                for h in range(H):
                    qh = q_v[e, pl.ds(h * DH, DH)]
                    kh = kv_v[e, pl.ds(h * DH, DH)]
                    score = jnp.where(lanes == off + h, jnp.sum(qh * kh),
                                      score)
            alpha16 = jnp.exp(score * 0.25 + ep_v[j, :])
            drows = e0 + lax.shift_right_logical(lanes, 2 + 1)
            dcols = D + (lanes & 7)
            plsc.store_scatter(msg_v, [drows, dcols], alpha16)
            for off, e in ((0, e0), (8, e0 + 1)):
                for h in range(H):
                    a = alpha16[off + h]
                    vh = kv_v[e, pl.ds(D + h * DH, DH)]
                    msg_v[e, pl.ds(h * DH, DH)] = a * vh
            return carry2
        lax.fori_loop(0, CH // 2, _pair, 0)

    _fetch_idx(0, 0)
    _wait_idx(0)
    _start_gather(0)
    _fetch_idx(1, 1)

    def _step(u, carry):
        for b in range(2):
            t = 2 * u + b
            pltpu.make_async_copy(q_hbm.at[dsts[b]], qs[b], semq[b]).wait()
            pltpu.make_async_copy(kv_hbm.at[srcs[b]], kvs[b], semk[b]).wait()

            @pl.when(t + 1 < NCHUNK)
            def _():
                _wait_idx(1 - b)
                _start_gather(1 - b)

            @pl.when(t >= 2)
            def _():
                pltpu.make_async_copy(
                    msgs[b], acc.at[dscat[b]], semsc[b]).wait()

            _compute(b)
            for o in (0, 16, CH - L):
                dscat[b][pl.ds(o, L)] = dsts[b][pl.ds(o, L)]
            pltpu.async_copy(msgs[b], acc.at[dscat[b]], semsc[b], add=True)

            @pl.when(t + 2 < NCHUNK)
            def _():
                _fetch_idx(t + 2, b)
        return carry
    lax.fori_loop(0, NCHUNK // 2, _step, 0)

    for b in range(2):
        pltpu.make_async_copy(msgs[b], acc.at[dscat[b]], semsc[b]).wait()
    plsc.subcore_barrier()
    pltpu.sync_copy(acc.at[pl.ds(s * RPS, RPS)],
                    out_hbm.at[c, pl.ds(s * RPS, RPS)])


def _edge_phase(q, kv, eproj, src, dst):
    mesh = plsc.VectorSubcoreMesh(core_axis_name="c", subcore_axis_name="s")
    run = pl.kernel(
        _sc_edge_body, mesh=mesh,
        compiler_params=pltpu.CompilerParams(
            needs_layout_passes=False, use_tc_tiling_on_sc=False),
        out_type=jax.ShapeDtypeStruct((NC, ACC_R, ROW), jnp.float32),
        scratch_types=(
            [pltpu.VMEM((CH,), jnp.int32)] * 6
            + [pltpu.VMEM((CH, D), jnp.float32)] * 2
            + [pltpu.VMEM((CH, 2 * D), jnp.float32)] * 2
            + [pltpu.VMEM((CH // 2, 2 * H), jnp.float32)] * 2
            + [pltpu.VMEM((CH, ROW), jnp.float32)] * 2
            + [pltpu.VMEM((ZB, ROW), jnp.float32)]
            + [pltpu.VMEM_SHARED((ACC_R, ROW), jnp.float32)]
            + [pltpu.SemaphoreType.DMA] * 8
        ),
    )
    return run(q, kv, eproj.reshape(E // 2, 2 * H), src, dst)


def _epi_body(acc_ref, x_ref, wout_ref, bout_ref, wg1_ref, wg2_ref,
              bg_ref, g1_ref, b1n_ref, w1_ref, bb1_ref, w2_ref, bb2_ref,
              g2_ref, b2n_ref, out_ref):
    a = acc_ref[0] + acc_ref[1]
    xb = x_ref[...]
    num = a[:, :D]
    rep = (jnp.arange(ROW)[:, None] == (D + jnp.arange(D)[None, :] // DH)
           ).astype(jnp.float32)
    den_rep = jnp.dot(a, rep, preferred_element_type=jnp.float32)
    agg = num / (den_rep + 1e-20)
    out_lin = jnp.dot(agg, wout_ref[...],
                      preferred_element_type=jnp.float32) + bout_ref[...]
    gz = (jnp.dot(out_lin, wg1_ref[...], preferred_element_type=jnp.float32)
          + jnp.dot(xb, wg2_ref[...], preferred_element_type=jnp.float32)
          + bg_ref[...])
    g = jax.nn.sigmoid(gz)
    h = g * out_lin + (1.0 - g) * xb
    mu = jnp.mean(h, axis=-1, keepdims=True)
    var = jnp.mean((h - mu) ** 2, axis=-1, keepdims=True)
    y = (h - mu) * jax.lax.rsqrt(var + 1e-5) * g1_ref[...] + b1n_ref[...]
    z = jnp.dot(y, w1_ref[...], preferred_element_type=jnp.float32) + bb1_ref[...]
    z = z * jax.nn.sigmoid(z)
    y2 = jnp.dot(z, w2_ref[...], preferred_element_type=jnp.float32) + bb2_ref[...]
    s = y + y2
    mu2 = jnp.mean(s, axis=-1, keepdims=True)
    var2 = jnp.mean((s - mu2) ** 2, axis=-1, keepdims=True)
    out_ref[...] = ((s - mu2) * jax.lax.rsqrt(var2 + 1e-5) * g2_ref[...]
                    + b2n_ref[...])


def _epilogue(acc, x, Wout, bout, Wg, bg, gamma1, beta1, W1, b1, W2, b2,
              gamma2, beta2):
    blk = 1000
    grid = (N // blk,)
    Wg1 = Wg[:D]
    Wg2 = Wg[D:]
    row = lambda i: (i, 0)
    full = lambda r, c: pl.BlockSpec((r, c), lambda i: (0, 0))
    vec = lambda c: pl.BlockSpec((1, c), lambda i: (0, 0))
    DFF = W1.shape[1]
    return pl.pallas_call(
        _epi_body,
        grid=grid,
        in_specs=[
            pl.BlockSpec((NC, blk, ROW), lambda i: (0, i, 0)),
            pl.BlockSpec((blk, D), row),
            full(D, D), vec(D), full(D, D), full(D, D), vec(D),
            vec(D), vec(D), full(D, DFF), vec(DFF), full(DFF, D), vec(D),
            vec(D), vec(D),
        ],
        out_specs=pl.BlockSpec((blk, D), row),
        out_shape=jax.ShapeDtypeStruct((N, D), jnp.float32),
    )(acc, x, Wout, bout.reshape(1, D), Wg1, Wg2, bg.reshape(1, D),
      gamma1.reshape(1, D), beta1.reshape(1, D), W1, b1.reshape(1, DFF),
      W2, b2.reshape(1, D), gamma2.reshape(1, D), beta2.reshape(1, D))


def kernel(x, edge_feat, edge_index, Wq, Wk, Wv, We, Wout, bout, Wg, bg,
           gamma1, beta1, W1, b1, W2, b2, gamma2, beta2):
    src = edge_index[0]
    dst = edge_index[1]
    q, kv = _qkv(x, Wq, Wk, Wv)
    eproj = _eproj(edge_feat, We)
    acc = _edge_phase(q, kv, eproj, src, dst)
    return _epilogue(acc, x, Wout, bout, Wg, bg, gamma1, beta1,
                     W1, b1, W2, b2, gamma2, beta2)


# speedup vs baseline: 5.6522x; 1.1022x over previous
import functools
import math

import jax
import jax.numpy as jnp
from jax import lax
from jax.experimental import pallas as pl
from jax.experimental.pallas import tpu as pltpu
from jax.experimental.pallas import tpu_sc as plsc

N = 10000
E = 320000
D = 128
H = 8
DH = 16

NC = 2
NS = 16
L = 16
NW = NC * NS
EPW = E // NW
CH = 40
NCHUNK = EPW // CH
ACC_R = ((N + NS * L - 1) // (NS * L)) * NS * L
RPS = ACC_R // NS
ZB = 4
ROW = 136


def _qkv_body(x_ref, wq_ref, wk_ref, wv_ref, q_ref, kv_ref):
    xb = x_ref[...]
    q_ref[...] = jnp.dot(xb, wq_ref[...],
                         preferred_element_type=jnp.float32) * (1.0 / 4.0)
    kv_ref[:, :D] = jnp.dot(xb, wk_ref[...], preferred_element_type=jnp.float32)
    kv_ref[:, D:] = jnp.dot(xb, wv_ref[...], preferred_element_type=jnp.float32)


def _qkv(x, Wq, Wk, Wv):
    blk = 1000
    grid = (N // blk,)
    return pl.pallas_call(
        _qkv_body,
        grid=grid,
        in_specs=[
            pl.BlockSpec((blk, D), lambda i: (i, 0)),
            pl.BlockSpec((D, D), lambda i: (0, 0)),
            pl.BlockSpec((D, D), lambda i: (0, 0)),
            pl.BlockSpec((D, D), lambda i: (0, 0)),
        ],
        out_specs=[
            pl.BlockSpec((blk, D), lambda i: (i, 0)),
            pl.BlockSpec((blk, 2 * D), lambda i: (i, 0)),
        ],
        out_shape=[
            jax.ShapeDtypeStruct((N, D), jnp.float32),
            jax.ShapeDtypeStruct((N, 2 * D), jnp.float32),
        ],
    )(x, Wq, Wk, Wv)


def _eproj_body(ef_ref, we_ref, out_ref):
    out_ref[...] = jnp.dot(ef_ref[...], we_ref[...],
                           preferred_element_type=jnp.float32)


def _eproj(edge_feat, We):
    ef8 = edge_feat.reshape(E // 8, 8 * 16)
    we_bd = jnp.zeros((8 * 16, 8 * H), jnp.float32)
    for j in range(8):
        we_bd = we_bd.at[j * 16:(j + 1) * 16, j * H:(j + 1) * H].set(We)
    blk = 4000
    grid = (E // 8 // blk,)
    out = pl.pallas_call(
        _eproj_body,
        grid=grid,
        in_specs=[
            pl.BlockSpec((blk, 128), lambda i: (i, 0)),
            pl.BlockSpec((128, 64), lambda i: (0, 0)),
        ],
        out_specs=pl.BlockSpec((blk, 64), lambda i: (i, 0)),
        out_shape=jax.ShapeDtypeStruct((E // 8, 64), jnp.float32),
    )(ef8, we_bd)
    return out.reshape(E, H)


def _sc_edge_body(q_hbm, kv_hbm, ep_hbm, src_hbm, dst_hbm, out_hbm,
                  s0, s1, d0, d1, dc0, dc1, q0, q1, kv0, kv1, e0v, e1v,
                  m0, m1, zb_v, acc, sq0, sq1, sk0, sk1, ss0, ss1,
                  si0, si1):
    c = lax.axis_index("c")
    s = lax.axis_index("s")
    wid = s * NC + c
    ebase = wid * EPW

    srcs = (s0, s1)
    dsts = (d0, d1)
    dscat = (dc0, dc1)
    qs = (q0, q1)
    kvs = (kv0, kv1)
    eps = (e0v, e1v)
    msgs = (m0, m1)
    semq = (sq0, sq1)
    semk = (sk0, sk1)
    semsc = (ss0, ss1)
    semi = (si0, si1)

    zero = jnp.zeros((L,), jnp.float32)
    for i in range(ZB):
        for j in range(ROW // L):
            zb_v[i, pl.ds(j * L, L)] = zero
        zb_v[i, pl.ds(ROW - L, L)] = zero

    def _zero_acc(t, carry):
        pltpu.sync_copy(zb_v, acc.at[pl.ds(s * RPS + t * ZB, ZB)])
        return carry
    lax.fori_loop(0, RPS // ZB, _zero_acc, 0)
    plsc.subcore_barrier()

    lanes = lax.iota(jnp.int32, L)

    def _fetch_idx(t, b):
        base = ebase + t * CH
        pltpu.async_copy(src_hbm.at[pl.ds(base, CH)], srcs[b], semi[b])
        pltpu.async_copy(dst_hbm.at[pl.ds(base, CH)], dsts[b], semi[b])
        pltpu.async_copy(
            ep_hbm.at[pl.ds(wid * (EPW // 2) + t * (CH // 2), CH // 2)],
            eps[b], semi[b])

    def _wait_idx(b):
        pltpu.make_async_copy(src_hbm.at[pl.ds(0, CH)], srcs[b],
                              semi[b]).wait()
        pltpu.make_async_copy(dst_hbm.at[pl.ds(0, CH)], dsts[b],
                              semi[b]).wait()
        pltpu.make_async_copy(ep_hbm.at[pl.ds(0, CH // 2)], eps[b],
                              semi[b]).wait()

    def _start_gather(b):
        pltpu.async_copy(q_hbm.at[dsts[b]], qs[b], semq[b])
        pltpu.async_copy(kv_hbm.at[srcs[b]], kvs[b], semk[b])

    def _compute(b):
        q_v, kv_v, ep_v, msg_v = qs[b], kvs[b], eps[b], msgs[b]

        @plsc.parallel_loop(0, CH // 2, 1, unroll=2)
        def _pair(j):
            e0 = 2 * j
            score = jnp.zeros((L,), jnp.float32)
            for off, e in ((0, e0), (8, e0 + 1)):
                for h in range(H):
                    qh = q_v[e, pl.ds(h * DH, DH)]
                    kh = kv_v[e, pl.ds(h * DH, DH)]
                    score = jnp.where(lanes == off + h, jnp.sum(qh * kh),
                                      score)
            alpha16 = jnp.exp(score + ep_v[j, :])
            drows = e0 + lax.shift_right_logical(lanes, 2 + 1)
            dcols = D + (lanes & 7)
            plsc.store_scatter(msg_v, [drows, dcols], alpha16)
            for off, e in ((0, e0), (8, e0 + 1)):
                for h in range(H):
                    a = alpha16[off + h]
                    vh = kv_v[e, pl.ds(D + h * DH, DH)]
                    msg_v[e, pl.ds(h * DH, DH)] = a * vh

    _fetch_idx(0, 0)
    _wait_idx(0)
    _start_gather(0)
    _fetch_idx(1, 1)

    def _step(u, carry):
        for b in range(2):
            t = 2 * u + b
            pltpu.make_async_copy(q_hbm.at[dsts[b]], qs[b], semq[b]).wait()
            pltpu.make_async_copy(kv_hbm.at[srcs[b]], kvs[b], semk[b]).wait()

            @pl.when(t + 1 < NCHUNK)
            def _():
                _wait_idx(1 - b)
                _start_gather(1 - b)

            @pl.when(t >= 2)
            def _():
                pltpu.make_async_copy(
                    msgs[b], acc.at[dscat[b]], semsc[b]).wait()

            _compute(b)
            for o in (0, 16, CH - L):
                dscat[b][pl.ds(o, L)] = dsts[b][pl.ds(o, L)]
            pltpu.async_copy(msgs[b], acc.at[dscat[b]], semsc[b], add=True)

            @pl.when(t + 2 < NCHUNK)
            def _():
                _fetch_idx(t + 2, b)
        return carry
    lax.fori_loop(0, NCHUNK // 2, _step, 0)

    for b in range(2):
        pltpu.make_async_copy(msgs[b], acc.at[dscat[b]], semsc[b]).wait()
    plsc.subcore_barrier()
    pltpu.sync_copy(acc.at[pl.ds(s * RPS, RPS)],
                    out_hbm.at[c, pl.ds(s * RPS, RPS)])


def _edge_phase(q, kv, eproj, src, dst):
    mesh = plsc.VectorSubcoreMesh(core_axis_name="c", subcore_axis_name="s")
    run = pl.kernel(
        _sc_edge_body, mesh=mesh,
        compiler_params=pltpu.CompilerParams(
            needs_layout_passes=False, use_tc_tiling_on_sc=False),
        out_type=jax.ShapeDtypeStruct((NC, ACC_R, ROW), jnp.float32),
        scratch_types=(
            [pltpu.VMEM((CH,), jnp.int32)] * 6
            + [pltpu.VMEM((CH, D), jnp.float32)] * 2
            + [pltpu.VMEM((CH, 2 * D), jnp.float32)] * 2
            + [pltpu.VMEM((CH // 2, 2 * H), jnp.float32)] * 2
            + [pltpu.VMEM((CH, ROW), jnp.float32)] * 2
            + [pltpu.VMEM((ZB, ROW), jnp.float32)]
            + [pltpu.VMEM_SHARED((ACC_R, ROW), jnp.float32)]
            + [pltpu.SemaphoreType.DMA] * 8
        ),
    )
    return run(q, kv, eproj.reshape(E // 2, 2 * H), src, dst)


def _epi_body(acc_ref, x_ref, wout_ref, bout_ref, wg1_ref, wg2_ref,
              bg_ref, g1_ref, b1n_ref, w1_ref, bb1_ref, w2_ref, bb2_ref,
              g2_ref, b2n_ref, out_ref):
    a = acc_ref[0] + acc_ref[1]
    xb = x_ref[...]
    num = a[:, :D]
    rep = (jnp.arange(ROW)[:, None] == (D + jnp.arange(D)[None, :] // DH)
           ).astype(jnp.float32)
    den_rep = jnp.dot(a, rep, preferred_element_type=jnp.float32)
    agg = num / (den_rep + 1e-20)
    out_lin = jnp.dot(agg, wout_ref[...],
                      preferred_element_type=jnp.float32) + bout_ref[...]
    gz = (jnp.dot(out_lin, wg1_ref[...], preferred_element_type=jnp.float32)
          + jnp.dot(xb, wg2_ref[...], preferred_element_type=jnp.float32)
          + bg_ref[...])
    g = jax.nn.sigmoid(gz)
    h = g * out_lin + (1.0 - g) * xb
    mu = jnp.mean(h, axis=-1, keepdims=True)
    var = jnp.mean((h - mu) ** 2, axis=-1, keepdims=True)
    y = (h - mu) * jax.lax.rsqrt(var + 1e-5) * g1_ref[...] + b1n_ref[...]
    z = jnp.dot(y, w1_ref[...], preferred_element_type=jnp.float32) + bb1_ref[...]
    z = z * jax.nn.sigmoid(z)
    y2 = jnp.dot(z, w2_ref[...], preferred_element_type=jnp.float32) + bb2_ref[...]
    s = y + y2
    mu2 = jnp.mean(s, axis=-1, keepdims=True)
    var2 = jnp.mean((s - mu2) ** 2, axis=-1, keepdims=True)
    out_ref[...] = ((s - mu2) * jax.lax.rsqrt(var2 + 1e-5) * g2_ref[...]
                    + b2n_ref[...])


def _epilogue(acc, x, Wout, bout, Wg, bg, gamma1, beta1, W1, b1, W2, b2,
              gamma2, beta2):
    blk = 1000
    grid = (N // blk,)
    Wg1 = Wg[:D]
    Wg2 = Wg[D:]
    row = lambda i: (i, 0)
    full = lambda r, c: pl.BlockSpec((r, c), lambda i: (0, 0))
    vec = lambda c: pl.BlockSpec((1, c), lambda i: (0, 0))
    DFF = W1.shape[1]
    return pl.pallas_call(
        _epi_body,
        grid=grid,
        in_specs=[
            pl.BlockSpec((NC, blk, ROW), lambda i: (0, i, 0)),
            pl.BlockSpec((blk, D), row),
            full(D, D), vec(D), full(D, D), full(D, D), vec(D),
            vec(D), vec(D), full(D, DFF), vec(DFF), full(DFF, D), vec(D),
            vec(D), vec(D),
        ],
        out_specs=pl.BlockSpec((blk, D), row),
        out_shape=jax.ShapeDtypeStruct((N, D), jnp.float32),
    )(acc, x, Wout, bout.reshape(1, D), Wg1, Wg2, bg.reshape(1, D),
      gamma1.reshape(1, D), beta1.reshape(1, D), W1, b1.reshape(1, DFF),
      W2, b2.reshape(1, D), gamma2.reshape(1, D), beta2.reshape(1, D))


def kernel(x, edge_feat, edge_index, Wq, Wk, Wv, We, Wout, bout, Wg, bg,
           gamma1, beta1, W1, b1, W2, b2, gamma2, beta2):
    src = edge_index[0]
    dst = edge_index[1]
    q, kv = _qkv(x, Wq, Wk, Wv)
    eproj = _eproj(edge_feat, We)
    acc = _edge_phase(q, kv, eproj, src, dst)
    return _epilogue(acc, x, Wout, bout, Wg, bg, gamma1, beta1,
                     W1, b1, W2, b2, gamma2, beta2)


# speedup vs baseline: 7.8906x; 1.3960x over previous
import functools
import math

import jax
import jax.numpy as jnp
from jax import lax
from jax.experimental import pallas as pl
from jax.experimental.pallas import tpu as pltpu
from jax.experimental.pallas import tpu_sc as plsc

N = 10000
E = 320000
D = 128
H = 8
DH = 16

NC = 2
NS = 16
L = 16
NW = NC * NS
EPW = E // NW
CH = 40
NCHUNK = EPW // CH
ACC_R = ((N + NS * L - 1) // (NS * L)) * NS * L
RPS = ACC_R // NS
ZB = 4
ROW = 136


def _qkv_body(x_ref, wq_ref, wk_ref, wv_ref, q_ref, kv_ref):
    xb = x_ref[...]
    q_ref[...] = jnp.dot(xb, wq_ref[...],
                         preferred_element_type=jnp.float32) * (1.0 / 4.0)
    kv_ref[:, :D] = jnp.dot(xb, wk_ref[...], preferred_element_type=jnp.float32)
    kv_ref[:, D:] = jnp.dot(xb, wv_ref[...], preferred_element_type=jnp.float32)


def _qkv(x, Wq, Wk, Wv):
    blk = 1000
    grid = (N // blk,)
    return pl.pallas_call(
        _qkv_body,
        grid=grid,
        in_specs=[
            pl.BlockSpec((blk, D), lambda i: (i, 0)),
            pl.BlockSpec((D, D), lambda i: (0, 0)),
            pl.BlockSpec((D, D), lambda i: (0, 0)),
            pl.BlockSpec((D, D), lambda i: (0, 0)),
        ],
        out_specs=[
            pl.BlockSpec((blk, D), lambda i: (i, 0)),
            pl.BlockSpec((blk, 2 * D), lambda i: (i, 0)),
        ],
        out_shape=[
            jax.ShapeDtypeStruct((N, D), jnp.float32),
            jax.ShapeDtypeStruct((N, 2 * D), jnp.float32),
        ],
    )(x, Wq, Wk, Wv)


def _eproj_body(ef_ref, we_ref, out_ref):
    out_ref[...] = jnp.dot(ef_ref[...], we_ref[...],
                           preferred_element_type=jnp.float32)


def _eproj(edge_feat, We):
    ef8 = edge_feat.reshape(E // 8, 8 * 16)
    we_bd = jnp.zeros((8 * 16, 8 * H), jnp.float32)
    for j in range(8):
        we_bd = we_bd.at[j * 16:(j + 1) * 16, j * H:(j + 1) * H].set(We)
    blk = 4000
    grid = (E // 8 // blk,)
    out = pl.pallas_call(
        _eproj_body,
        grid=grid,
        in_specs=[
            pl.BlockSpec((blk, 128), lambda i: (i, 0)),
            pl.BlockSpec((128, 64), lambda i: (0, 0)),
        ],
        out_specs=pl.BlockSpec((blk, 64), lambda i: (i, 0)),
        out_shape=jax.ShapeDtypeStruct((E // 8, 64), jnp.float32),
    )(ef8, we_bd)
    return out.reshape(E, H)


def _sc_edge_body(q_hbm, kv_hbm, ep_hbm, src_hbm, dst_hbm, out_hbm,
                  s0, s1, d0, d1, dc0, dc1, q0, q1, kv0, kv1, e0v, e1v,
                  m0, m1, zb_v, acc, sq0, sq1, sk0, sk1, ss0, ss1,
                  si0, si1):
    c = lax.axis_index("c")
    s = lax.axis_index("s")
    wid = s * NC + c
    ebase = wid * EPW

    srcs = (s0, s1)
    dsts = (d0, d1)
    dscat = (dc0, dc1)
    qs = (q0, q1)
    kvs = (kv0, kv1)
    eps = (e0v, e1v)
    msgs = (m0, m1)
    semq = (sq0, sq1)
    semk = (sk0, sk1)
    semsc = (ss0, ss1)
    semi = (si0, si1)

    zero = jnp.zeros((L,), jnp.float32)
    for i in range(ZB):
        for j in range(ROW // L):
            zb_v[i, pl.ds(j * L, L)] = zero
        zb_v[i, pl.ds(ROW - L, L)] = zero

    def _zero_acc(t, carry):
        pltpu.sync_copy(zb_v, acc.at[pl.ds(s * RPS + t * ZB, ZB)])
        return carry
    lax.fori_loop(0, RPS // ZB, _zero_acc, 0)
    plsc.subcore_barrier()

    lanes = lax.iota(jnp.int32, L)

    def _fetch_idx(t, b):
        base = ebase + t * CH
        pltpu.async_copy(src_hbm.at[pl.ds(base, CH)], srcs[b], semi[b])
        pltpu.async_copy(dst_hbm.at[pl.ds(base, CH)], dsts[b], semi[b])
        pltpu.async_copy(
            ep_hbm.at[pl.ds(wid * (EPW // 2) + t * (CH // 2), CH // 2)],
            eps[b], semi[b])

    def _wait_idx(b):
        pltpu.make_async_copy(src_hbm.at[pl.ds(0, CH)], srcs[b],
                              semi[b]).wait()
        pltpu.make_async_copy(dst_hbm.at[pl.ds(0, CH)], dsts[b],
                              semi[b]).wait()
        pltpu.make_async_copy(ep_hbm.at[pl.ds(0, CH // 2)], eps[b],
                              semi[b]).wait()

    def _start_gather(b):
        pltpu.async_copy(q_hbm.at[dsts[b]], qs[b], semq[b])
        pltpu.async_copy(kv_hbm.at[srcs[b]], kvs[b], semk[b])

    def _compute(b):
        q_v, kv_v, ep_v, msg_v = qs[b], kvs[b], eps[b], msgs[b]

        @plsc.parallel_loop(0, CH // 2, 1, unroll=4)
        def _pair(j):
            e0 = 2 * j
            score = jnp.zeros((L,), jnp.float32)
            for off, e in ((0, e0), (8, e0 + 1)):
                for h in range(H):
                    qh = q_v[e, pl.ds(h * DH, DH)]
                    kh = kv_v[e, pl.ds(h * DH, DH)]
                    score = jnp.where(lanes == off + h, jnp.sum(qh * kh),
                                      score)
            alpha16 = jnp.exp(score + ep_v[j, :])
            drows = e0 + lax.shift_right_logical(lanes, 2 + 1)
            dcols = D + (lanes & 7)
            plsc.store_scatter(msg_v, [drows, dcols], alpha16)
            for off, e in ((0, e0), (8, e0 + 1)):
                for h in range(H):
                    a = alpha16[off + h]
                    vh = kv_v[e, pl.ds(D + h * DH, DH)]
                    msg_v[e, pl.ds(h * DH, DH)] = a * vh

    _fetch_idx(0, 0)
    _wait_idx(0)
    _start_gather(0)
    _fetch_idx(1, 1)

    def _step(u, carry):
        for b in range(2):
            t = 2 * u + b
            pltpu.make_async_copy(q_hbm.at[dsts[b]], qs[b], semq[b]).wait()
            pltpu.make_async_copy(kv_hbm.at[srcs[b]], kvs[b], semk[b]).wait()

            @pl.when(t + 1 < NCHUNK)
            def _():
                _wait_idx(1 - b)
                _start_gather(1 - b)

            @pl.when(t >= 2)
            def _():
                pltpu.make_async_copy(
                    msgs[b], acc.at[dscat[b]], semsc[b]).wait()

            _compute(b)
            for o in (0, 16, CH - L):
                dscat[b][pl.ds(o, L)] = dsts[b][pl.ds(o, L)]
            pltpu.async_copy(msgs[b], acc.at[dscat[b]], semsc[b], add=True)

            @pl.when(t + 2 < NCHUNK)
            def _():
                _fetch_idx(t + 2, b)
        return carry
    lax.fori_loop(0, NCHUNK // 2, _step, 0)

    for b in range(2):
        pltpu.make_async_copy(msgs[b], acc.at[dscat[b]], semsc[b]).wait()
    plsc.subcore_barrier()
    pltpu.sync_copy(acc.at[pl.ds(s * RPS, RPS)],
                    out_hbm.at[c, pl.ds(s * RPS, RPS)])


def _edge_phase(q, kv, eproj, src, dst):
    mesh = plsc.VectorSubcoreMesh(core_axis_name="c", subcore_axis_name="s")
    run = pl.kernel(
        _sc_edge_body, mesh=mesh,
        compiler_params=pltpu.CompilerParams(
            needs_layout_passes=False, use_tc_tiling_on_sc=False),
        out_type=jax.ShapeDtypeStruct((NC, ACC_R, ROW), jnp.float32),
        scratch_types=(
            [pltpu.VMEM((CH,), jnp.int32)] * 6
            + [pltpu.VMEM((CH, D), jnp.float32)] * 2
            + [pltpu.VMEM((CH, 2 * D), jnp.float32)] * 2
            + [pltpu.VMEM((CH // 2, 2 * H), jnp.float32)] * 2
            + [pltpu.VMEM((CH, ROW), jnp.float32)] * 2
            + [pltpu.VMEM((ZB, ROW), jnp.float32)]
            + [pltpu.VMEM_SHARED((ACC_R, ROW), jnp.float32)]
            + [pltpu.SemaphoreType.DMA] * 8
        ),
    )
    return run(q, kv, eproj.reshape(E // 2, 2 * H), src, dst)


def _epi_body(acc_ref, x_ref, wout_ref, bout_ref, wg1_ref, wg2_ref,
              bg_ref, g1_ref, b1n_ref, w1_ref, bb1_ref, w2_ref, bb2_ref,
              g2_ref, b2n_ref, out_ref):
    a = acc_ref[0] + acc_ref[1]
    xb = x_ref[...]
    num = a[:, :D]
    rep = (jnp.arange(ROW)[:, None] == (D + jnp.arange(D)[None, :] // DH)
           ).astype(jnp.float32)
    den_rep = jnp.dot(a, rep, preferred_element_type=jnp.float32)
    agg = num / (den_rep + 1e-20)
    out_lin = jnp.dot(agg, wout_ref[...],
                      preferred_element_type=jnp.float32) + bout_ref[...]
    gz = (jnp.dot(out_lin, wg1_ref[...], preferred_element_type=jnp.float32)
          + jnp.dot(xb, wg2_ref[...], preferred_element_type=jnp.float32)
          + bg_ref[...])
    g = jax.nn.sigmoid(gz)
    h = g * out_lin + (1.0 - g) * xb
    mu = jnp.mean(h, axis=-1, keepdims=True)
    var = jnp.mean((h - mu) ** 2, axis=-1, keepdims=True)
    y = (h - mu) * jax.lax.rsqrt(var + 1e-5) * g1_ref[...] + b1n_ref[...]
    z = jnp.dot(y, w1_ref[...], preferred_element_type=jnp.float32) + bb1_ref[...]
    z = z * jax.nn.sigmoid(z)
    y2 = jnp.dot(z, w2_ref[...], preferred_element_type=jnp.float32) + bb2_ref[...]
    s = y + y2
    mu2 = jnp.mean(s, axis=-1, keepdims=True)
    var2 = jnp.mean((s - mu2) ** 2, axis=-1, keepdims=True)
    out_ref[...] = ((s - mu2) * jax.lax.rsqrt(var2 + 1e-5) * g2_ref[...]
                    + b2n_ref[...])


def _epilogue(acc, x, Wout, bout, Wg, bg, gamma1, beta1, W1, b1, W2, b2,
              gamma2, beta2):
    blk = 1000
    grid = (N // blk,)
    Wg1 = Wg[:D]
    Wg2 = Wg[D:]
    row = lambda i: (i, 0)
    full = lambda r, c: pl.BlockSpec((r, c), lambda i: (0, 0))
    vec = lambda c: pl.BlockSpec((1, c), lambda i: (0, 0))
    DFF = W1.shape[1]
    return pl.pallas_call(
        _epi_body,
        grid=grid,
        in_specs=[
            pl.BlockSpec((NC, blk, ROW), lambda i: (0, i, 0)),
            pl.BlockSpec((blk, D), row),
            full(D, D), vec(D), full(D, D), full(D, D), vec(D),
            vec(D), vec(D), full(D, DFF), vec(DFF), full(DFF, D), vec(D),
            vec(D), vec(D),
        ],
        out_specs=pl.BlockSpec((blk, D), row),
        out_shape=jax.ShapeDtypeStruct((N, D), jnp.float32),
    )(acc, x, Wout, bout.reshape(1, D), Wg1, Wg2, bg.reshape(1, D),
      gamma1.reshape(1, D), beta1.reshape(1, D), W1, b1.reshape(1, DFF),
      W2, b2.reshape(1, D), gamma2.reshape(1, D), beta2.reshape(1, D))


def kernel(x, edge_feat, edge_index, Wq, Wk, Wv, We, Wout, bout, Wg, bg,
           gamma1, beta1, W1, b1, W2, b2, gamma2, beta2):
    src = edge_index[0]
    dst = edge_index[1]
    q, kv = _qkv(x, Wq, Wk, Wv)
    eproj = _eproj(edge_feat, We)
    acc = _edge_phase(q, kv, eproj, src, dst)
    return _epilogue(acc, x, Wout, bout, Wg, bg, gamma1, beta1,
                     W1, b1, W2, b2, gamma2, beta2)
